# Initial kernel scaffold; baseline (speedup 1.0000x reference)
import functools

import jax
import jax.numpy as jnp
from jax import lax
from jax.experimental import pallas as pl
from jax.experimental.pallas import tpu as pltpu
from jax.experimental.pallas import tpu_sc as plsc

N_NODES = 10000
D = 128
N_EDGES = 320000
NC = 2
NS = 16
NW = NC * NS
EDGES_PER_W = N_EDGES // NW
CHUNK = 80
CHUNKS_PER_W = EDGES_PER_W // CHUNK
ROWS_PER_SUB = N_NODES // NS
ZROWS = 125
LANES = 16


def _mm_body(x_ref, w_ref, o_ref):
    o_ref[...] = jnp.dot(x_ref[...], w_ref[...],
                         preferred_element_type=jnp.float32)


def _matmul(x, W):
    M, K = x.shape
    _, N = W.shape
    BM = 2000
    return pl.pallas_call(
        _mm_body,
        grid=(M // BM,),
        in_specs=[pl.BlockSpec((BM, K), lambda i: (i, 0)),
                  pl.BlockSpec((K, N), lambda i: (0, 0))],
        out_specs=pl.BlockSpec((BM, N), lambda i: (i, 0)),
        out_shape=jax.ShapeDtypeStruct((M, N), jnp.float32),
    )(x, W)


def _add_body(p_ref, o_ref):
    o_ref[...] = p_ref[0] + p_ref[1]


def _sum_partials(p):
    _, M, N = p.shape
    BM = 2000
    return pl.pallas_call(
        _add_body,
        grid=(M // BM,),
        in_specs=[pl.BlockSpec((2, BM, N), lambda i: (0, i, 0))],
        out_specs=pl.BlockSpec((BM, N), lambda i: (i, 0)),
        out_shape=jax.ShapeDtypeStruct((M, N), jnp.float32),
    )(p)


def _sc_body(support_hbm, src_hbm, dst_hbm, w_hbm, out_hbm,
             src_v, dst_v, w_v, rows_v, zbuf, acc, sem):
    c = lax.axis_index("c")
    s = lax.axis_index("s")

    def _zrow(r, _):
        for j in range(D // LANES):
            zbuf[r, pl.ds(j * LANES, LANES)] = jnp.zeros((LANES,), jnp.float32)
        return 0
    lax.fori_loop(0, ZROWS, _zrow, 0)
    for z in range(ROWS_PER_SUB // ZROWS):
        pltpu.sync_copy(zbuf, acc.at[pl.ds(s * ROWS_PER_SUB + z * ZROWS, ZROWS)])
    plsc.subcore_barrier()

    wid = c * NS + s
    base = wid * EDGES_PER_W

    def _chunk(i, _):
        off = base + i * CHUNK
        pltpu.sync_copy(src_hbm.at[pl.ds(off, CHUNK)], src_v)
        pltpu.sync_copy(dst_hbm.at[pl.ds(off, CHUNK)], dst_v)
        pltpu.sync_copy(w_hbm.at[pl.ds(off, CHUNK)], w_v)
        pltpu.async_copy(support_hbm.at[src_v], rows_v, sem).wait()

        def _scale(e, _):
            wb = plsc.load_gather(w_v, [jnp.full((LANES,), e, jnp.int32)])
            for j in range(D // LANES):
                sl = pl.ds(j * LANES, LANES)
                rows_v[e, sl] = rows_v[e, sl] * wb
            return 0
        lax.fori_loop(0, CHUNK, _scale, 0)

        pltpu.sync_copy(rows_v, acc.at[dst_v], add=True)
        return 0

    lax.fori_loop(0, CHUNKS_PER_W, _chunk, 0)
    plsc.subcore_barrier()

    pltpu.sync_copy(acc.at[pl.ds(s * ROWS_PER_SUB, ROWS_PER_SUB)],
                    out_hbm.at[c, pl.ds(s * ROWS_PER_SUB, ROWS_PER_SUB)])


def _sc_spmm(support, src, dst, w):
    mesh = plsc.VectorSubcoreMesh(core_axis_name="c", subcore_axis_name="s")
    kfn = functools.partial(
        pl.kernel,
        out_type=jax.ShapeDtypeStruct((NC, N_NODES, D), jnp.float32),
        mesh=mesh,
        scratch_types=[
            pltpu.VMEM((CHUNK,), jnp.int32),
            pltpu.VMEM((CHUNK,), jnp.int32),
            pltpu.VMEM((CHUNK,), jnp.float32),
            pltpu.VMEM((CHUNK, D), jnp.float32),
            pltpu.VMEM((ZROWS, D), jnp.float32),
            pltpu.VMEM_SHARED((N_NODES, D), jnp.float32),
            pltpu.SemaphoreType.DMA,
        ],
    )(_sc_body)
    return kfn(support, src, dst, w)


def kernel(x, edge_index, edge_weight, W):
    support = _matmul(x, W)
    dst = edge_index[0].astype(jnp.int32)
    src = edge_index[1].astype(jnp.int32)
    partials = _sc_spmm(support, src, dst, edge_weight)
    return _sum_partials(partials)


# speedup vs baseline: 4.0348x; 4.0348x over previous
import functools

import jax
import jax.numpy as jnp
from jax import lax
from jax.experimental import pallas as pl
from jax.experimental.pallas import tpu as pltpu
from jax.experimental.pallas import tpu_sc as plsc

N_NODES = 10000
D = 128
N_EDGES = 320000
NC = 2
NS = 16
NW = NC * NS
EDGES_PER_W = N_EDGES // NW
CHUNK = 80
CHUNKS_PER_W = EDGES_PER_W // CHUNK
ACC_ROWS = 10240
ROWS_PER_SUB = ACC_ROWS // NS
ZROWS = 128
LANES = 16


def _mm_body(x_ref, w_ref, o_ref):
    o_ref[...] = jnp.dot(x_ref[...], w_ref[...],
                         preferred_element_type=jnp.float32)


def _matmul(x, W):
    M, K = x.shape
    _, N = W.shape
    BM = 2000
    return pl.pallas_call(
        _mm_body,
        grid=(M // BM,),
        in_specs=[pl.BlockSpec((BM, K), lambda i: (i, 0)),
                  pl.BlockSpec((K, N), lambda i: (0, 0))],
        out_specs=pl.BlockSpec((BM, N), lambda i: (i, 0)),
        out_shape=jax.ShapeDtypeStruct((M, N), jnp.float32),
    )(x, W)


def _add_body(p_ref, o_ref):
    o_ref[...] = p_ref[0] + p_ref[1]


def _sum_partials(p):
    M, N = N_NODES, D
    BM = 2000
    return pl.pallas_call(
        _add_body,
        grid=(M // BM,),
        in_specs=[pl.BlockSpec((2, BM, N), lambda i: (0, i, 0))],
        out_specs=pl.BlockSpec((BM, N), lambda i: (i, 0)),
        out_shape=jax.ShapeDtypeStruct((M, N), jnp.float32),
    )(p)


def _sc_body(support_hbm, src_hbm, dst_hbm, w_hbm, out_hbm,
             src_v, dst_v, w_v, rows_v, zbuf, acc, sem):
    c = lax.axis_index("c")
    s = lax.axis_index("s")

    def _zrow(r, _):
        for j in range(D // LANES):
            zbuf[r, pl.ds(j * LANES, LANES)] = jnp.zeros((LANES,), jnp.float32)
        return 0
    lax.fori_loop(0, ZROWS, _zrow, 0)
    for z in range(ROWS_PER_SUB // ZROWS):
        pltpu.sync_copy(zbuf, acc.at[pl.ds(s * ROWS_PER_SUB + z * ZROWS, ZROWS)])
    plsc.subcore_barrier()

    wid = c * NS + s
    base = wid * EDGES_PER_W

    def _chunk(i, _):
        off = base + i * CHUNK
        pltpu.sync_copy(src_hbm.at[pl.ds(off, CHUNK)], src_v)
        pltpu.sync_copy(dst_hbm.at[pl.ds(off, CHUNK)], dst_v)
        pltpu.sync_copy(w_hbm.at[pl.ds(off, CHUNK)], w_v)
        pltpu.async_copy(support_hbm.at[src_v], rows_v, sem).wait()

        def _scale(e, _):
            wb = plsc.load_gather(w_v, [jnp.full((LANES,), e, jnp.int32)])
            for j in range(D // LANES):
                sl = pl.ds(j * LANES, LANES)
                rows_v[e, sl] = rows_v[e, sl] * wb
            return 0
        lax.fori_loop(0, CHUNK, _scale, 0)

        pltpu.sync_copy(rows_v, acc.at[dst_v], add=True)
        return 0

    lax.fori_loop(0, CHUNKS_PER_W, _chunk, 0)
    plsc.subcore_barrier()

    pltpu.sync_copy(acc.at[pl.ds(s * ROWS_PER_SUB, ROWS_PER_SUB)],
                    out_hbm.at[c, pl.ds(s * ROWS_PER_SUB, ROWS_PER_SUB)])


def _sc_spmm(support, src, dst, w):
    mesh = plsc.VectorSubcoreMesh(core_axis_name="c", subcore_axis_name="s")
    kfn = functools.partial(
        pl.kernel,
        out_type=jax.ShapeDtypeStruct((NC, ACC_ROWS, D), jnp.float32),
        mesh=mesh,
        compiler_params=pltpu.CompilerParams(needs_layout_passes=False),
        scratch_types=[
            pltpu.VMEM((CHUNK,), jnp.int32),
            pltpu.VMEM((CHUNK,), jnp.int32),
            pltpu.VMEM((CHUNK,), jnp.float32),
            pltpu.VMEM((CHUNK, D), jnp.float32),
            pltpu.VMEM((ZROWS, D), jnp.float32),
            pltpu.VMEM_SHARED((ACC_ROWS, D), jnp.float32),
            pltpu.SemaphoreType.DMA,
        ],
    )(_sc_body)
    return kfn(support, src, dst, w)


def kernel(x, edge_index, edge_weight, W):
    support = _matmul(x, W)
    dst = edge_index[0].astype(jnp.int32)
    src = edge_index[1].astype(jnp.int32)
    partials = _sc_spmm(support, src, dst, edge_weight)
    return _sum_partials(partials)


# speedup vs baseline: 9.5801x; 2.3744x over previous
import functools

import jax
import jax.numpy as jnp
from jax import lax
from jax.experimental import pallas as pl
from jax.experimental.pallas import tpu as pltpu
from jax.experimental.pallas import tpu_sc as plsc

N_NODES = 10000
D = 128
N_EDGES = 320000
NC = 2
NS = 16
NW = NC * NS
EDGES_PER_W = N_EDGES // NW
CHUNK = 80
NCHUNK = EDGES_PER_W // CHUNK
NBUF = 2
ACC_ROWS = 10240
ROWS_PER_SUB = ACC_ROWS // NS
ZROWS = 16
LANES = 16
G16 = CHUNK // LANES


def _mm_body(x_ref, w_ref, o_ref):
    o_ref[...] = jnp.dot(x_ref[...], w_ref[...],
                         preferred_element_type=jnp.float32)


def _matmul(x, W):
    M, K = x.shape
    _, N = W.shape
    BM = 2000
    return pl.pallas_call(
        _mm_body,
        grid=(M // BM,),
        in_specs=[pl.BlockSpec((BM, K), lambda i: (i, 0)),
                  pl.BlockSpec((K, N), lambda i: (0, 0))],
        out_specs=pl.BlockSpec((BM, N), lambda i: (i, 0)),
        out_shape=jax.ShapeDtypeStruct((M, N), jnp.float32),
    )(x, W)


def _add_body(p_ref, o_ref):
    o_ref[...] = p_ref[0] + p_ref[1]


def _sum_partials(p):
    M, N = N_NODES, D
    BM = 2000
    return pl.pallas_call(
        _add_body,
        grid=(M // BM,),
        in_specs=[pl.BlockSpec((2, BM, N), lambda i: (0, i, 0))],
        out_specs=pl.BlockSpec((BM, N), lambda i: (i, 0)),
        out_shape=jax.ShapeDtypeStruct((M, N), jnp.float32),
    )(p)


def _sc_body(support_hbm, src_hbm, dst_hbm, w_hbm, out_hbm,
             src_v, dst_v, w_v, rows_v, zbuf, acc, semg, sems):
    c = lax.axis_index("c")
    s = lax.axis_index("s")
    wid = c * NS + s
    base = wid * EDGES_PER_W

    pltpu.sync_copy(src_hbm.at[pl.ds(base, EDGES_PER_W)], src_v)
    pltpu.sync_copy(dst_hbm.at[pl.ds(base, EDGES_PER_W)], dst_v)

    def _zrow(r, _):
        for j in range(D // LANES):
            zbuf[r, pl.ds(j * LANES, LANES)] = jnp.zeros((LANES,), jnp.float32)
        return 0
    lax.fori_loop(0, ZROWS, _zrow, 0)
    for z in range(ROWS_PER_SUB // ZROWS):
        pltpu.sync_copy(zbuf, acc.at[pl.ds(s * ROWS_PER_SUB + z * ZROWS, ZROWS)])
    plsc.subcore_barrier()

    def _gather(i, b):
        pltpu.async_copy(w_hbm.at[pl.ds(base + i * CHUNK, CHUNK)],
                         w_v.at[b], semg.at[b])
        for k in range(G16):
            idx = src_v[pl.ds(i * CHUNK + k * LANES, LANES)]
            pltpu.async_copy(support_hbm.at[idx],
                             rows_v.at[b, pl.ds(k * LANES, LANES)],
                             semg.at[b])

    def _wait_gather(b):
        pltpu.make_async_copy(w_hbm.at[pl.ds(0, CHUNK)], w_v.at[b],
                              semg.at[b]).wait()
        pltpu.make_async_copy(support_hbm.at[pl.ds(0, CHUNK)], rows_v.at[b],
                              semg.at[b]).wait()

    def _scale(i, b):
        rb = rows_v.at[b]
        wv = w_v.at[b]

        def _edges(u, _):
            for t in range(4):
                e = u * 4 + t
                wb = plsc.load_gather(wv, [jnp.full((LANES,), e, jnp.int32)])
                for j in range(D // LANES):
                    sl = pl.ds(j * LANES, LANES)
                    rb[e, sl] = rb[e, sl] * wb
            return 0
        lax.fori_loop(0, CHUNK // 4, _edges, 0)

    def _scatter(i, b):
        for k in range(G16):
            idx = dst_v[pl.ds(i * CHUNK + k * LANES, LANES)]
            pltpu.async_copy(rows_v.at[b, pl.ds(k * LANES, LANES)],
                             acc.at[idx], sems.at[b], add=True)

    def _wait_scatter(b):
        pltpu.make_async_copy(rows_v.at[b], acc.at[pl.ds(0, CHUNK)],
                              sems.at[b]).wait()

    for b in range(NBUF):
        _gather(b, b)

    def _body(p, _):
        for b in range(NBUF):
            i = p * NBUF + b
            _wait_gather(b)
            _scale(i, b)
            _scatter(i, b)
            _wait_scatter(b)

            @pl.when(i + NBUF < NCHUNK)
            def _():
                _gather(i + NBUF, b)
        return 0

    lax.fori_loop(0, (NCHUNK - 1) // NBUF, _body, 0)

    tb = (NCHUNK - 1) % NBUF
    _wait_gather(tb)
    _scale(NCHUNK - 1, tb)
    _scatter(NCHUNK - 1, tb)
    _wait_scatter(tb)

    plsc.subcore_barrier()

    pltpu.sync_copy(acc.at[pl.ds(s * ROWS_PER_SUB, ROWS_PER_SUB)],
                    out_hbm.at[c, pl.ds(s * ROWS_PER_SUB, ROWS_PER_SUB)])


def _sc_spmm(support, src, dst, w):
    mesh = plsc.VectorSubcoreMesh(core_axis_name="c", subcore_axis_name="s")
    kfn = functools.partial(
        pl.kernel,
        out_type=jax.ShapeDtypeStruct((NC, ACC_ROWS, D), jnp.float32),
        mesh=mesh,
        compiler_params=pltpu.CompilerParams(needs_layout_passes=False),
        scratch_types=[
            pltpu.VMEM((EDGES_PER_W,), jnp.int32),
            pltpu.VMEM((EDGES_PER_W,), jnp.int32),
            pltpu.VMEM((NBUF, CHUNK), jnp.float32),
            pltpu.VMEM((NBUF, CHUNK, D), jnp.float32),
            pltpu.VMEM((ZROWS, D), jnp.float32),
            pltpu.VMEM_SHARED((ACC_ROWS, D), jnp.float32),
            pltpu.SemaphoreType.DMA((NBUF,)),
            pltpu.SemaphoreType.DMA((NBUF,)),
        ],
    )(_sc_body)
    return kfn(support, src, dst, w)


def kernel(x, edge_index, edge_weight, W):
    support = _matmul(x, W)
    dst = edge_index[0].astype(jnp.int32)
    src = edge_index[1].astype(jnp.int32)
    partials = _sc_spmm(support, src, dst, edge_weight)
    return _sum_partials(partials)
